# Initial kernel scaffold; baseline (speedup 1.0000x reference)
import jax
import jax.numpy as jnp
from jax.experimental import pallas as pl


def kernel(inputs, embeddings):
    raise NotImplementedError("write your pallas kernel here")


# speedup vs baseline: 5.1809x; 5.1809x over previous
import functools

import jax
import jax.numpy as jnp
from jax import lax
from jax.experimental import pallas as pl
from jax.experimental.pallas import tpu as pltpu
from jax.experimental.pallas import tpu_sc as plsc

DIM = 128
NUM_CORES = 2
NUM_SUBCORES = 16
NUM_WORKERS = NUM_CORES * NUM_SUBCORES
CHUNK = 128


def _make_gather(total, vocab):
    assert total % (NUM_WORKERS * CHUNK) == 0
    per_worker = total // NUM_WORKERS
    n_chunks = per_worker // CHUNK

    mesh = plsc.VectorSubcoreMesh(
        core_axis_name="c",
        subcore_axis_name="s",
        num_cores=NUM_CORES,
        num_subcores=NUM_SUBCORES,
    )

    @functools.partial(
        pl.kernel,
        out_type=jax.ShapeDtypeStruct((total, DIM), jnp.float32),
        mesh=mesh,
        scratch_types=[
            pltpu.VMEM((CHUNK,), jnp.int32),
            pltpu.VMEM((CHUNK, DIM), jnp.float32),
            pltpu.SemaphoreType.DMA,
        ],
    )
    def gather(idx_hbm, table_hbm, out_hbm, idx_v, rows_v, sem):
        wid = lax.axis_index("s") * NUM_CORES + lax.axis_index("c")
        base = wid * per_worker

        def chunk_body(i):
            off = base + i * CHUNK
            pltpu.sync_copy(idx_hbm.at[pl.ds(off, CHUNK)], idx_v)
            pltpu.async_copy(table_hbm.at[idx_v], rows_v, sem).wait()
            pltpu.sync_copy(rows_v, out_hbm.at[pl.ds(off, CHUNK)])

        pl.loop(0, n_chunks)(chunk_body)

    return gather


def kernel(inputs, embeddings):
    batch, hist = inputs.shape
    idx = inputs.reshape(-1).astype(jnp.int32)
    out = _make_gather(idx.shape[0], embeddings.shape[0])(idx, embeddings)
    return out.reshape(batch, hist, DIM)


# speedup vs baseline: 9.2335x; 1.7822x over previous
import functools

import jax
import jax.numpy as jnp
from jax import lax
from jax.experimental import pallas as pl
from jax.experimental.pallas import tpu as pltpu
from jax.experimental.pallas import tpu_sc as plsc

DIM = 128
NUM_CORES = 2
NUM_SUBCORES = 16
NUM_WORKERS = NUM_CORES * NUM_SUBCORES
CHUNK = 128
NBUF = 4
LAG = 2


def _make_gather(total, vocab):
    assert total % (NUM_WORKERS * CHUNK) == 0
    per_worker = total // NUM_WORKERS
    n_chunks = per_worker // CHUNK
    assert n_chunks % NBUF == 0 and n_chunks >= 2 * NBUF

    mesh = plsc.VectorSubcoreMesh(
        core_axis_name="c",
        subcore_axis_name="s",
        num_cores=NUM_CORES,
        num_subcores=NUM_SUBCORES,
    )

    @functools.partial(
        pl.kernel,
        out_type=jax.ShapeDtypeStruct((total, DIM), jnp.float32),
        mesh=mesh,
        scratch_types=[
            pltpu.VMEM((n_chunks, CHUNK), jnp.int32),
            [pltpu.VMEM((CHUNK, DIM), jnp.float32) for _ in range(NBUF)],
            [pltpu.SemaphoreType.DMA for _ in range(NBUF)],
            [pltpu.SemaphoreType.DMA for _ in range(NBUF)],
        ],
    )
    def gather(idx_hbm, table_hbm, out_hbm, idx_v, rows, gsem, osem):
        wid = lax.axis_index("s") * NUM_CORES + lax.axis_index("c")
        base = wid * per_worker

        pltpu.sync_copy(idx_hbm.at[pl.ds(wid * n_chunks, n_chunks)], idx_v)

        def start_gather(t, b):
            return pltpu.make_async_copy(
                table_hbm.at[idx_v.at[t]], rows[b], gsem[b])

        def start_out(t, b):
            return pltpu.make_async_copy(
                rows[b], out_hbm.at[pl.ds(base + t * CHUNK, CHUNK)], osem[b])

        for b in range(NBUF):
            start_gather(b, b).start()
            if b >= LAG:
                u, bu = b - LAG, (b - LAG) % NBUF
                start_gather(u, bu).wait()
                start_out(u, bu).start()

        def outer(g):
            for b in range(NBUF):
                t = g * NBUF + b
                start_out(t - NBUF, b).wait()
                start_gather(t, b).start()
                u, bu = t - LAG, (b - LAG) % NBUF
                start_gather(u, bu).wait()
                start_out(u, bu).start()

        pl.loop(1, n_chunks // NBUF)(outer)

        for u in range(n_chunks - LAG, n_chunks):
            bu = u % NBUF
            start_gather(u, bu).wait()
            start_out(u, bu).start()
        for b in range(NBUF):
            start_out(n_chunks - NBUF + b, b).wait()

    return gather


def kernel(inputs, embeddings):
    batch, hist = inputs.shape
    idx = inputs.reshape(-1).astype(jnp.int32)
    total = idx.shape[0]
    idx2d = idx.reshape(total // CHUNK, CHUNK)
    out = _make_gather(total, embeddings.shape[0])(idx2d, embeddings)
    return out.reshape(batch, hist, DIM)


# speedup vs baseline: 9.2828x; 1.0053x over previous
import functools

import jax
import jax.numpy as jnp
from jax import lax
from jax.experimental import pallas as pl
from jax.experimental.pallas import tpu as pltpu
from jax.experimental.pallas import tpu_sc as plsc

DIM = 128
NUM_CORES = 2
NUM_SUBCORES = 16
NUM_WORKERS = NUM_CORES * NUM_SUBCORES
CHUNK = 128
NBUF = 5
LAG = 2


def _make_gather(total, vocab):
    assert total % (NUM_WORKERS * CHUNK) == 0
    per_worker = total // NUM_WORKERS
    n_chunks = per_worker // CHUNK
    assert n_chunks % NBUF == 0 and n_chunks >= 2 * NBUF

    mesh = plsc.VectorSubcoreMesh(
        core_axis_name="c",
        subcore_axis_name="s",
        num_cores=NUM_CORES,
        num_subcores=NUM_SUBCORES,
    )

    @functools.partial(
        pl.kernel,
        out_type=jax.ShapeDtypeStruct((total, DIM), jnp.float32),
        mesh=mesh,
        scratch_types=[
            pltpu.VMEM((n_chunks, CHUNK), jnp.int32),
            [pltpu.VMEM((CHUNK, DIM), jnp.float32) for _ in range(NBUF)],
            [pltpu.SemaphoreType.DMA for _ in range(NBUF)],
            [pltpu.SemaphoreType.DMA for _ in range(NBUF)],
        ],
    )
    def gather(idx_hbm, table_hbm, out_hbm, idx_v, rows, gsem, osem):
        wid = lax.axis_index("s") * NUM_CORES + lax.axis_index("c")
        base = wid * per_worker

        pltpu.sync_copy(idx_hbm.at[pl.ds(wid * n_chunks, n_chunks)], idx_v)

        def start_gather(t, b):
            return pltpu.make_async_copy(
                table_hbm.at[idx_v.at[t]], rows[b], gsem[b])

        def start_out(t, b):
            return pltpu.make_async_copy(
                rows[b], out_hbm.at[pl.ds(base + t * CHUNK, CHUNK)], osem[b])

        for b in range(NBUF):
            start_gather(b, b).start()
            if b >= LAG:
                u, bu = b - LAG, (b - LAG) % NBUF
                start_gather(u, bu).wait()
                start_out(u, bu).start()

        def outer(g):
            for b in range(NBUF):
                t = g * NBUF + b
                start_out(t - NBUF, b).wait()
                start_gather(t, b).start()
                u, bu = t - LAG, (b - LAG) % NBUF
                start_gather(u, bu).wait()
                start_out(u, bu).start()

        pl.loop(1, n_chunks // NBUF)(outer)

        for u in range(n_chunks - LAG, n_chunks):
            bu = u % NBUF
            start_gather(u, bu).wait()
            start_out(u, bu).start()
        for b in range(NBUF):
            start_out(n_chunks - NBUF + b, b).wait()

    return gather


def kernel(inputs, embeddings):
    batch, hist = inputs.shape
    idx = inputs.reshape(-1).astype(jnp.int32)
    total = idx.shape[0]
    idx2d = idx.reshape(total // CHUNK, CHUNK)
    out = _make_gather(total, embeddings.shape[0])(idx2d, embeddings)
    return out.reshape(batch, hist, DIM)
